# Initial kernel scaffold; baseline (speedup 1.0000x reference)
import functools

import jax
import jax.numpy as jnp
from jax import lax
from jax.experimental import pallas as pl
from jax.experimental.pallas import tpu as pltpu
from jax.experimental.pallas import tpu_sc as plsc

HIDDEN = 64
VOCAB = 2016
SCALER = 7

NC = 2
NS = 16
NW = NC * NS
L = 16

TOTAL = 4096 * 200
PER_W = TOTAL // NW
BLK = 512
GRP = 128
NBLK = PER_W // BLK


def _table_body(emb_ref, w_ref, b_ref, out_ref):
    t = lax.dot_general(
        emb_ref[:], w_ref[:], (((1,), (1,)), ((), ())),
        preferred_element_type=jnp.float32,
    )
    t = t + b_ref[:]
    out_ref[:] = jnp.concatenate([t[:, 1:], jnp.sin(t[:, :1])], axis=1)


def _build_table(emb, W, b):
    return pl.pallas_call(
        _table_body,
        out_shape=jax.ShapeDtypeStruct((VOCAB, HIDDEN), jnp.float32),
    )(emb, W, b.reshape(1, HIDDEN))


def _gather_body(tod_hbm, dow_hbm, table_hbm, out_hbm,
                 tod_v, dow_v, idx_v, rows_v, gsem):
    wid = lax.axis_index("s") * NC + lax.axis_index("c")
    base = wid * PER_W

    def block(g, carry):
        off = base + g * BLK
        pltpu.sync_copy(tod_hbm.at[pl.ds(off, BLK)], tod_v)
        pltpu.sync_copy(dow_hbm.at[pl.ds(off, BLK)], dow_v)
        for i in range(BLK // L):
            s = pl.ds(i * L, L)
            j, c = divmod(i, GRP // L)
            idx_v[j, pl.ds(c * L, L)] = tod_v[s] * SCALER + dow_v[s]
        copies = [
            pltpu.async_copy(table_hbm.at[idx_v.at[j]],
                             rows_v.at[pl.ds(j * GRP, GRP)], gsem)
            for j in range(BLK // GRP)
        ]
        for c in copies:
            c.wait()
        pltpu.sync_copy(rows_v, out_hbm.at[pl.ds(off, BLK)])
        return carry

    lax.fori_loop(0, NBLK, block, 0)


@functools.partial(
    pl.kernel,
    mesh=plsc.VectorSubcoreMesh(core_axis_name="c", subcore_axis_name="s"),
    out_type=jax.ShapeDtypeStruct((TOTAL, HIDDEN), jnp.float32),
    scratch_types=[
        pltpu.VMEM((BLK,), jnp.int32),
        pltpu.VMEM((BLK,), jnp.int32),
        pltpu.VMEM((BLK // GRP, GRP), jnp.int32),
        pltpu.VMEM((BLK, HIDDEN), jnp.float32),
        pltpu.SemaphoreType.DMA,
    ],
)
def _sc_gather(tod_hbm, dow_hbm, table_hbm, out_hbm,
               tod_v, dow_v, idx_v, rows_v, gsem):
    _gather_body(tod_hbm, dow_hbm, table_hbm, out_hbm,
                 tod_v, dow_v, idx_v, rows_v, gsem)


def kernel(tod, dow, emb, W, b):
    table = _build_table(emb, W, b)
    out = _sc_gather(tod.reshape(-1), dow.reshape(-1), table)
    return out.reshape(tod.shape + (HIDDEN,))


# speedup vs baseline: 6.8568x; 6.8568x over previous
import functools

import jax
import jax.numpy as jnp
from jax import lax
from jax.experimental import pallas as pl
from jax.experimental.pallas import tpu as pltpu
from jax.experimental.pallas import tpu_sc as plsc

HIDDEN = 64
VOCAB = 2016
SCALER = 7

NC = 2
NS = 16
NW = NC * NS
L = 16

TOTAL = 4096 * 200
PER_W = TOTAL // NW
BLK = 512
GRP = 128
NBLK = PER_W // BLK


def _table_body(emb_ref, w_ref, b_ref, out_ref):
    t = lax.dot_general(
        emb_ref[:], w_ref[:], (((1,), (1,)), ((), ())),
        preferred_element_type=jnp.float32,
    )
    t = t + b_ref[:]
    out_ref[:] = jnp.concatenate([t[:, 1:], jnp.sin(t[:, :1])], axis=1)


def _build_table(emb, W, b):
    return pl.pallas_call(
        _table_body,
        out_shape=jax.ShapeDtypeStruct((VOCAB, HIDDEN), jnp.float32),
    )(emb, W, b.reshape(1, HIDDEN))


def _gather_body(tod_hbm, dow_hbm, table_hbm, out_hbm,
                 tod_v, dow_v, idx_v, rows_v, table_sh, gsem):
    sid = lax.axis_index("s")
    wid = sid * NC + lax.axis_index("c")
    base = wid * PER_W

    @pl.when(sid == 0)
    def _():
        pltpu.sync_copy(table_hbm, table_sh)

    plsc.subcore_barrier()

    def block(g, carry):
        off = base + g * BLK
        pltpu.sync_copy(tod_hbm.at[pl.ds(off, BLK)], tod_v)
        pltpu.sync_copy(dow_hbm.at[pl.ds(off, BLK)], dow_v)
        for i in range(BLK // L):
            s = pl.ds(i * L, L)
            j, c = divmod(i, GRP // L)
            idx_v[j, pl.ds(c * L, L)] = tod_v[s] * SCALER + dow_v[s]
        copies = [
            pltpu.async_copy(table_sh.at[idx_v.at[j]],
                             rows_v.at[pl.ds(j * GRP, GRP)], gsem)
            for j in range(BLK // GRP)
        ]
        for c in copies:
            c.wait()
        pltpu.sync_copy(rows_v, out_hbm.at[pl.ds(off, BLK)])
        return carry

    lax.fori_loop(0, NBLK, block, 0)


@functools.partial(
    pl.kernel,
    mesh=plsc.VectorSubcoreMesh(core_axis_name="c", subcore_axis_name="s"),
    out_type=jax.ShapeDtypeStruct((TOTAL, HIDDEN), jnp.float32),
    scratch_types=[
        pltpu.VMEM((BLK,), jnp.int32),
        pltpu.VMEM((BLK,), jnp.int32),
        pltpu.VMEM((BLK // GRP, GRP), jnp.int32),
        pltpu.VMEM((BLK, HIDDEN), jnp.float32),
        pltpu.VMEM_SHARED((VOCAB, HIDDEN), jnp.float32),
        pltpu.SemaphoreType.DMA,
    ],
)
def _sc_gather(tod_hbm, dow_hbm, table_hbm, out_hbm,
               tod_v, dow_v, idx_v, rows_v, table_sh, gsem):
    _gather_body(tod_hbm, dow_hbm, table_hbm, out_hbm,
                 tod_v, dow_v, idx_v, rows_v, table_sh, gsem)


def kernel(tod, dow, emb, W, b):
    table = _build_table(emb, W, b)
    out = _sc_gather(tod.reshape(-1), dow.reshape(-1), table)
    return out.reshape(tod.shape + (HIDDEN,))
